# baseline (speedup 1.0000x reference)
import functools

import jax
import jax.numpy as jnp
from jax import lax
from jax.experimental import pallas as pl
from jax.experimental.pallas import tpu as pltpu

B, D, E, H, O, K = 2048, 768, 64, 64, 64, 2
BLK = 256
NBLK = B // BLK


def _fused_kernel(x_ref, wg_ref, bg_ref, w1_ref, b1_ref, w2_ref, b2_ref,
                  out_ref, scores_ref, idx_ref, ent_ref):
    blk = pl.program_id(0)
    xb = x_ref[...]

    scores = lax.dot_general(xb, wg_ref[...],
                             (((1,), (1,)), ((), ())))
    scores = scores + bg_ref[...][None, :]
    scores_ref[...] = scores

    m = jnp.max(scores, axis=1, keepdims=True)
    ex = jnp.exp(scores - m)
    z = jnp.sum(ex, axis=1, keepdims=True)
    probs = ex / z

    ent_rows = jnp.sum(-probs * jnp.log(probs + 1e-9), axis=1, keepdims=True)
    ent_blk = jnp.sum(ent_rows, axis=0, keepdims=True) / B

    @pl.when(blk == 0)
    def _():
        ent_ref[...] = ent_blk

    @pl.when(blk > 0)
    def _():
        ent_ref[...] += ent_blk

    iota = lax.broadcasted_iota(jnp.int32, (BLK, E), 1)
    p0 = jnp.max(probs, axis=1, keepdims=True)
    i0 = jnp.min(jnp.where(probs == p0, iota, E), axis=1, keepdims=True)
    pm = jnp.where(iota == i0, -jnp.inf, probs)
    p1 = jnp.max(pm, axis=1, keepdims=True)
    i1 = jnp.min(jnp.where(pm == p1, iota, E), axis=1, keepdims=True)
    idx_ref[...] = jnp.concatenate([i0, i1], axis=1)

    eab = jnp.exp(p1 - p0)
    nw0 = 1.0 / (1.0 + eab)
    nw1 = eab / (1.0 + eab)

    def body(e, acc):
        w1e = w1_ref[e]
        h = jnp.maximum(
            jnp.dot(xb, w1e, preferred_element_type=jnp.float32)
            + b1_ref[pl.ds(e, 1), :], 0.0)
        ye = (jnp.dot(h, w2_ref[e], preferred_element_type=jnp.float32)
              + b2_ref[pl.ds(e, 1), :])
        we = (jnp.where(i0 == e, nw0, 0.0)
              + jnp.where(i1 == e, nw1, 0.0))
        return acc + we * ye

    out_ref[...] = lax.fori_loop(0, E, body, jnp.zeros((BLK, O), jnp.float32))


def kernel(x, Wg, bg, W1, b1, W2, b2):
    out, scores, idx, ent = pl.pallas_call(
        _fused_kernel,
        grid=(NBLK,),
        in_specs=[
            pl.BlockSpec((BLK, D), lambda i: (i, 0)),
            pl.BlockSpec((E, D), lambda i: (0, 0)),
            pl.BlockSpec((E,), lambda i: (0,)),
            pl.BlockSpec((E, D, H), lambda i: (0, 0, 0)),
            pl.BlockSpec((E, H), lambda i: (0, 0)),
            pl.BlockSpec((E, H, O), lambda i: (0, 0, 0)),
            pl.BlockSpec((E, O), lambda i: (0, 0)),
        ],
        out_specs=[
            pl.BlockSpec((BLK, O), lambda i: (i, 0)),
            pl.BlockSpec((BLK, E), lambda i: (i, 0)),
            pl.BlockSpec((BLK, K), lambda i: (i, 0)),
            pl.BlockSpec((1, 1), lambda i: (0, 0)),
        ],
        out_shape=[
            jax.ShapeDtypeStruct((B, O), jnp.float32),
            jax.ShapeDtypeStruct((B, E), jnp.float32),
            jax.ShapeDtypeStruct((B, K), jnp.int32),
            jax.ShapeDtypeStruct((1, 1), jnp.float32),
        ],
    )(x, Wg, bg, W1, b1, W2, b2)
    return out, scores, idx, ent[0, 0]
